# Initial kernel scaffold; baseline (speedup 1.0000x reference)
import jax
import jax.numpy as jnp
from jax.experimental import pallas as pl


def kernel(emb, x, hash_idx, pull_idx):
    raise NotImplementedError("write your pallas kernel here")


# speedup vs baseline: 3.0437x; 3.0437x over previous
import functools

import jax
import jax.numpy as jnp
from jax import lax
from jax.experimental import pallas as pl
from jax.experimental.pallas import tpu as pltpu
from jax.experimental.pallas import tpu_sc as plsc

N_EMB = 1000000
B = 16384
D = 64
NC = 2
NS = 16
L = 16

MS_W = 62528
DUMMY = NS * MS_W
MSIZE = DUMMY + B
HB = B // NS
HCHUNK = HB // 128
PB = B // (NC * NS)
PCHUNK = PB // 128
FILL = 4096
MS_TAIL = MS_W - (MS_W // FILL) * FILL
ROUNDS = 3

_mesh = plsc.VectorSubcoreMesh(
    core_axis_name="c", subcore_axis_name="s", num_cores=NC, num_subcores=NS)


@functools.partial(
    pl.kernel,
    out_type=jax.ShapeDtypeStruct((B, D), jnp.float32),
    mesh=_mesh,
    compiler_params=pltpu.CompilerParams(use_tc_tiling_on_sc=False),
    scratch_types=dict(
        marker=pltpu.VMEM_SHARED((MSIZE,), jnp.int32),
        fbuf=pltpu.VMEM((FILL,), jnp.int32),
        hbuf=pltpu.VMEM((HCHUNK, 128), jnp.int32),
        jbuf=pltpu.VMEM((HCHUNK, 128), jnp.int32),
        wbuf=pltpu.VMEM((HCHUNK, 128), jnp.int32),
        sbuf=pltpu.VMEM((HCHUNK, 128), jnp.int32),
        pbuf=pltpu.VMEM((PCHUNK, 128), jnp.int32),
        mbuf=pltpu.VMEM((PCHUNK, 128), jnp.int32),
        erows=pltpu.VMEM((PB, D), jnp.float32),
        sem_ms=pltpu.SemaphoreType.DMA,
        sem_e=pltpu.SemaphoreType.DMA,
    ),
)
def _sc_push_pull(emb, x, hash2d, pull2d, out,
                  marker, fbuf, hbuf, jbuf, wbuf, sbuf, pbuf, mbuf,
                  erows, sem_ms, sem_e):
    cid = lax.axis_index("c")
    sid = lax.axis_index("s")
    tid = cid * NS + sid
    lanes = jnp.arange(L, dtype=jnp.int32)

    def _fill(i, _):
        fbuf[pl.ds(i * L, L)] = jnp.full((L,), -1, jnp.int32)
        return _
    lax.fori_loop(0, FILL // L, _fill, 0)
    ms_copies = [
        pltpu.async_copy(
            fbuf, marker.at[pl.ds(sid * MS_W + k * FILL, FILL)], sem_ms)
        for k in range(MS_W // FILL)
    ]
    ms_copies.append(pltpu.async_copy(
        fbuf.at[pl.ds(0, MS_TAIL)],
        marker.at[pl.ds(sid * MS_W + (MS_W // FILL) * FILL, MS_TAIL)],
        sem_ms))

    pltpu.sync_copy(pull2d.at[pl.ds(tid * PCHUNK, PCHUNK)], pbuf)
    e_copies = [
        pltpu.async_copy(emb.at[pbuf.at[k]], erows.at[pl.ds(k * 128, 128)],
                         sem_e)
        for k in range(PCHUNK)
    ]

    pltpu.sync_copy(hash2d.at[pl.ds(sid * HCHUNK, HCHUNK)], hbuf)
    for k in range(HCHUNK):
        for c in range(128 // L):
            jbuf[k, pl.ds(c * L, L)] = lanes + (sid * HB + k * 128 + c * L)

    for cp in ms_copies:
        cp.wait()
    plsc.subcore_barrier()

    for k in range(HCHUNK):
        pltpu.sync_copy(jbuf.at[k], marker.at[hbuf.at[k]])
    plsc.subcore_barrier()

    for _ in range(ROUNDS):
        for k in range(HCHUNK):
            pltpu.sync_copy(marker.at[hbuf.at[k]], wbuf.at[k])
        for k in range(HCHUNK):
            for c in range(128 // L):
                sl = pl.ds(c * L, L)
                j = jbuf[k, sl]
                w = wbuf[k, sl]
                h = hbuf[k, sl]
                sbuf[k, sl] = jnp.where(j > w, h, DUMMY + j)
        plsc.subcore_barrier()
        for k in range(HCHUNK):
            pltpu.sync_copy(jbuf.at[k], marker.at[sbuf.at[k]])
        plsc.subcore_barrier()

    for k in range(PCHUNK):
        pltpu.sync_copy(marker.at[pbuf.at[k]], mbuf.at[k])

    for cp in e_copies:
        cp.wait()

    for k in range(PCHUNK):
        def _patch(g, carry, k=k):
            mv = mbuf[k, pl.ds(g * L, L)]
            for l in range(L):
                m = mv[l]
                @pl.when(m >= 0)
                def _do(m=m, l=l):
                    pltpu.sync_copy(x.at[pl.ds(m, 1)],
                                    erows.at[pl.ds(k * 128 + g * L + l, 1)])
            return carry
        lax.fori_loop(0, 128 // L, _patch, 0)

    pltpu.sync_copy(erows, out.at[pl.ds(tid * PB, PB)])


def kernel(emb, x, hash_idx, pull_idx):
    hash2d = hash_idx.reshape(B // 128, 128)
    pull2d = pull_idx.reshape(B // 128, 128)
    return _sc_push_pull(emb, x, hash2d, pull2d)
